# baseline (speedup 1.0000x reference)
import functools
import math

import jax
import jax.numpy as jnp
import numpy as np
from jax import lax
from jax.experimental import pallas as pl
from jax.experimental.pallas import tpu as pltpu

try:
    from jax.experimental.pallas import tpu_sc as plsc
except ImportError:
    plsc = None

EMB = 128
TEMB = 16
PE_MAX = 512
EPS = 1e-12


def _sc_gather(table, idx3d, n_tokens):
    info = plsc.get_sparse_core_info()
    nw = info.num_cores * info.num_subcores
    chunks_per_w = (n_tokens // 128) // nw
    per_w = chunks_per_w * 128
    mesh = plsc.VectorSubcoreMesh(core_axis_name="c", subcore_axis_name="s")

    nbuf = 5
    assert chunks_per_w % nbuf == 0, chunks_per_w

    @functools.partial(
        pl.kernel,
        mesh=mesh,
        out_type=jax.ShapeDtypeStruct((n_tokens, EMB), jnp.float32),
        scratch_types=[
            pltpu.VMEM((chunks_per_w, 128), jnp.int32),
            pltpu.VMEM((nbuf, 128, EMB), jnp.float32),
            pltpu.SemaphoreType.DMA((nbuf,)),
            pltpu.SemaphoreType.DMA((nbuf,)),
        ],
    )
    def gather_k(table_hbm, idx_hbm, out_hbm, idx_v, rows_v, sg, ss):
        wid = lax.axis_index("s") * info.num_cores + lax.axis_index("c")
        pltpu.sync_copy(idx_hbm.at[wid], idx_v)
        base = wid * per_w

        def g_start(j, b):
            pltpu.make_async_copy(table_hbm.at[idx_v.at[j]], rows_v.at[b],
                                  sg.at[b]).start()

        def g_wait(b):
            pltpu.make_async_copy(table_hbm.at[idx_v.at[0]], rows_v.at[b],
                                  sg.at[b]).wait()

        def s_start(j, b):
            pltpu.make_async_copy(rows_v.at[b],
                                  out_hbm.at[pl.ds(base + j * 128, 128)],
                                  ss.at[b]).start()

        def s_wait(b):
            pltpu.make_async_copy(rows_v.at[b], out_hbm.at[pl.ds(base, 128)],
                                  ss.at[b]).wait()

        for b in range(nbuf):
            g_start(b, b)

        def outer(t0, carry):
            for b in range(nbuf):
                j = t0 * nbuf + b
                g_wait(b)
                s_start(j, b)
                s_wait(b)
                nj = j + nbuf

                @pl.when(nj < chunks_per_w)
                def _():
                    g_start(nj, b)
            return carry

        lax.fori_loop(0, chunks_per_w // nbuf, outer, 0)

    return gather_k(table, idx3d)


def _fast_sin(x):
    inv_2pi = 0.15915494309189535
    c1 = 6.28125
    c2 = 0.0019353071795864769
    k = jnp.round(x * inv_2pi)
    r = (x - k * c1) - k * c2
    r2 = r * r
    p = 1.0 / 6227020800.0
    p = p * r2 - 1.0 / 39916800.0
    p = p * r2 + 1.0 / 362880.0
    p = p * r2 - 1.0 / 5040.0
    p = p * r2 + 1.0 / 120.0
    p = p * r2 - 1.0 / 6.0
    p = p * r2 + 1.0
    return r * p


def _tc_body(g_ref, ints_ref, wc_ref, wf_ref, a_ref, b48_ref, c_ref, d_ref,
             vt_ref, b_ref, gamma_ref, beta_ref, out_ref):
    ints = ints_ref[...]
    tsf = ints[:, 0:1].astype(jnp.float32)
    agef = ints[:, 1:2].astype(jnp.float32)
    normf = jnp.clip(ints[:, 2:3] - ints[:, 3:4], 0, PE_MAX - 1).astype(jnp.float32)
    arg = tsf * a_ref[...] + agef * b48_ref[...] + normf * c_ref[...] + d_ref[...]
    feat = _fast_sin(arg)
    acc = jnp.dot(g_ref[...], wc_ref[...], preferred_element_type=jnp.float32)
    acc = acc + jnp.dot(feat, wf_ref[...], preferred_element_type=jnp.float32)
    acc = acc + b_ref[...]
    x = jnp.tanh(acc)
    vs = ints[:, 4:5]
    seg = jnp.where(vs == 0, vt_ref[0:1, :],
                    jnp.where(vs == 1, vt_ref[1:2, :], vt_ref[2:3, :]))
    x = x + seg
    mu = jnp.mean(x, axis=-1, keepdims=True)
    var = jnp.mean(x * x, axis=-1, keepdims=True) - mu * mu
    out_ref[...] = ((x - mu) / jnp.sqrt(var + EPS)) * gamma_ref[...] + beta_ref[...]


def kernel(concept_ids, time_stamps, ages, visit_orders, visit_segments,
           concept_table, visit_table, w_time, phi_time, w_age, phi_age,
           pe, W, b, gamma, beta):
    B, L = concept_ids.shape
    BL = B * L
    nsplit = 2
    half = BL // nsplit

    idx4d = concept_ids.astype(jnp.int32).reshape(nsplit, 32, half // (32 * 128), 128)
    gathered = [_sc_gather(concept_table, idx4d[h], half) for h in range(nsplit)]

    i32 = jnp.int32
    ints = jnp.concatenate([
        time_stamps.astype(i32).reshape(BL, 1),
        ages.astype(i32).reshape(BL, 1),
        visit_orders.astype(i32).reshape(BL, 1),
        jnp.broadcast_to(visit_orders[:, 0:1], (B, L)).astype(i32).reshape(BL, 1),
        visit_segments.astype(i32).reshape(BL, 1),
        jnp.zeros((BL, 3), i32),
    ], axis=1)

    wc = W[:EMB]
    wf = jnp.concatenate([W[EMB:EMB + TEMB], W[EMB + TEMB:EMB + 2 * TEMB],
                          W[EMB + 2 * TEMB::2], W[EMB + 2 * TEMB + 1::2]], axis=0)
    div = np.exp(np.arange(0, TEMB, 2, dtype=np.float32)
                 * -(math.log(10000.0) / TEMB)).astype(np.float32)
    z8 = np.zeros(8, np.float32)
    z16 = np.zeros(16, np.float32)
    arow = jnp.concatenate([w_time[0], jnp.asarray(np.concatenate([z16, z8, z8]))]).reshape(1, 48)
    brow = jnp.concatenate([jnp.asarray(z16), w_age[0], jnp.asarray(np.concatenate([z8, z8]))]).reshape(1, 48)
    crow = jnp.asarray(np.concatenate([z16, z16, div, div])).reshape(1, 48)
    drow = jnp.concatenate([phi_time[0], phi_age[0],
                            jnp.asarray(np.concatenate([z8, np.full(8, math.pi / 2, np.float32)]))]).reshape(1, 48)

    T = 2048
    nb = half // T
    full = lambda shape: pl.BlockSpec(shape, lambda i: tuple(0 for _ in shape))

    def tc_call(g_half, ints_half):
        return pl.pallas_call(
            _tc_body,
            grid=(nb,),
            in_specs=[
                pl.BlockSpec((T, EMB), lambda i: (i, 0)),
                pl.BlockSpec((T, 8), lambda i: (i, 0)),
                full((EMB, EMB)),
                full((48, EMB)),
                full((1, 48)), full((1, 48)), full((1, 48)), full((1, 48)),
                full((3, EMB)),
                full((1, EMB)), full((1, EMB)), full((1, EMB)),
            ],
            out_specs=pl.BlockSpec((T, EMB), lambda i: (i, 0)),
            out_shape=jax.ShapeDtypeStruct((half, EMB), jnp.float32),
        )(g_half, ints_half, wc, wf, arow, brow, crow, drow,
          visit_table, b.reshape(1, EMB), gamma.reshape(1, EMB), beta.reshape(1, EMB))

    outs = [tc_call(gathered[h], ints[h * half:(h + 1) * half]) for h in range(nsplit)]
    return jnp.concatenate(outs, axis=0).reshape(B, L, EMB)


# speedup vs baseline: 1.1558x; 1.1558x over previous
import functools
import math

import jax
import jax.numpy as jnp
import numpy as np
from jax import lax
from jax.experimental import pallas as pl
from jax.experimental.pallas import tpu as pltpu

try:
    from jax.experimental.pallas import tpu_sc as plsc
except ImportError:
    plsc = None

EMB = 128
TEMB = 16
PE_MAX = 512
EPS = 1e-12


def _sc_gather(table, idx3d, n_tokens):
    info = plsc.get_sparse_core_info()
    nw = info.num_cores * info.num_subcores
    chunks_per_w = (n_tokens // 128) // nw
    per_w = chunks_per_w * 128
    mesh = plsc.VectorSubcoreMesh(core_axis_name="c", subcore_axis_name="s")

    nbuf = 5
    assert chunks_per_w % nbuf == 0, chunks_per_w

    @functools.partial(
        pl.kernel,
        mesh=mesh,
        out_type=jax.ShapeDtypeStruct((n_tokens, EMB), jnp.float32),
        scratch_types=[
            pltpu.VMEM((chunks_per_w, 128), jnp.int32),
            pltpu.VMEM((nbuf, 128, EMB), jnp.float32),
            pltpu.SemaphoreType.DMA((nbuf,)),
            pltpu.SemaphoreType.DMA((nbuf,)),
        ],
    )
    def gather_k(table_hbm, idx_hbm, out_hbm, idx_v, rows_v, sg, ss):
        wid = lax.axis_index("s") * info.num_cores + lax.axis_index("c")
        pltpu.sync_copy(idx_hbm.at[wid], idx_v)
        base = wid * per_w

        def g_start(j, b):
            pltpu.make_async_copy(table_hbm.at[idx_v.at[j]], rows_v.at[b],
                                  sg.at[b]).start()

        def g_wait(b):
            pltpu.make_async_copy(table_hbm.at[idx_v.at[0]], rows_v.at[b],
                                  sg.at[b]).wait()

        def s_start(j, b):
            pltpu.make_async_copy(rows_v.at[b],
                                  out_hbm.at[pl.ds(base + j * 128, 128)],
                                  ss.at[b]).start()

        def s_wait(b):
            pltpu.make_async_copy(rows_v.at[b], out_hbm.at[pl.ds(base, 128)],
                                  ss.at[b]).wait()

        for b in range(nbuf):
            g_start(b, b)

        def outer(t0, carry):
            for b in range(nbuf):
                j = t0 * nbuf + b
                g_wait(b)
                s_start(j, b)
                s_wait(b)
                nj = j + nbuf

                @pl.when(nj < chunks_per_w)
                def _():
                    g_start(nj, b)
            return carry

        lax.fori_loop(0, chunks_per_w // nbuf, outer, 0)

    return gather_k(table, idx3d)


def _fast_sin(x):
    inv_2pi = 0.15915494309189535
    c1 = 6.28125
    c2 = 0.0019353071795864769
    k = jnp.round(x * inv_2pi)
    r = (x - k * c1) - k * c2
    r2 = r * r
    p = 1.0 / 6227020800.0
    p = p * r2 - 1.0 / 39916800.0
    p = p * r2 + 1.0 / 362880.0
    p = p * r2 - 1.0 / 5040.0
    p = p * r2 + 1.0 / 120.0
    p = p * r2 - 1.0 / 6.0
    p = p * r2 + 1.0
    return r * p


def _tc_body(g_ref, ints_ref, wc_ref, wf_ref, a_ref, b48_ref, c_ref, d_ref,
             vt_ref, b_ref, gamma_ref, beta_ref, out_ref):
    ints = ints_ref[...]
    tsf = ints[:, 0:1].astype(jnp.float32)
    agef = ints[:, 1:2].astype(jnp.float32)
    normf = jnp.clip(ints[:, 2:3] - ints[:, 3:4], 0, PE_MAX - 1).astype(jnp.float32)
    arg = tsf * a_ref[...] + agef * b48_ref[...] + normf * c_ref[...] + d_ref[...]
    feat = _fast_sin(arg)
    acc = jnp.dot(g_ref[...], wc_ref[...], preferred_element_type=jnp.float32)
    acc = acc + jnp.dot(feat, wf_ref[...], preferred_element_type=jnp.float32)
    acc = acc + b_ref[...]
    x = jnp.tanh(acc)
    vs = ints[:, 4:5]
    seg = jnp.where(vs == 0, vt_ref[0:1, :],
                    jnp.where(vs == 1, vt_ref[1:2, :], vt_ref[2:3, :]))
    x = x + seg
    mu = jnp.mean(x, axis=-1, keepdims=True)
    var = jnp.mean(x * x, axis=-1, keepdims=True) - mu * mu
    out_ref[...] = ((x - mu) / jnp.sqrt(var + EPS)) * gamma_ref[...] + beta_ref[...]


def kernel(concept_ids, time_stamps, ages, visit_orders, visit_segments,
           concept_table, visit_table, w_time, phi_time, w_age, phi_age,
           pe, W, b, gamma, beta):
    B, L = concept_ids.shape
    BL = B * L
    nsplit = 1
    half = BL // nsplit

    idx4d = concept_ids.astype(jnp.int32).reshape(nsplit, 32, half // (32 * 128), 128)
    gathered = [_sc_gather(concept_table, idx4d[h], half) for h in range(nsplit)]

    i32 = jnp.int32
    ints = jnp.concatenate([
        time_stamps.astype(i32).reshape(BL, 1),
        ages.astype(i32).reshape(BL, 1),
        visit_orders.astype(i32).reshape(BL, 1),
        jnp.broadcast_to(visit_orders[:, 0:1], (B, L)).astype(i32).reshape(BL, 1),
        visit_segments.astype(i32).reshape(BL, 1),
        jnp.zeros((BL, 3), i32),
    ], axis=1)

    wc = W[:EMB]
    wf = jnp.concatenate([W[EMB:EMB + TEMB], W[EMB + TEMB:EMB + 2 * TEMB],
                          W[EMB + 2 * TEMB::2], W[EMB + 2 * TEMB + 1::2]], axis=0)
    div = np.exp(np.arange(0, TEMB, 2, dtype=np.float32)
                 * -(math.log(10000.0) / TEMB)).astype(np.float32)
    z8 = np.zeros(8, np.float32)
    z16 = np.zeros(16, np.float32)
    arow = jnp.concatenate([w_time[0], jnp.asarray(np.concatenate([z16, z8, z8]))]).reshape(1, 48)
    brow = jnp.concatenate([jnp.asarray(z16), w_age[0], jnp.asarray(np.concatenate([z8, z8]))]).reshape(1, 48)
    crow = jnp.asarray(np.concatenate([z16, z16, div, div])).reshape(1, 48)
    drow = jnp.concatenate([phi_time[0], phi_age[0],
                            jnp.asarray(np.concatenate([z8, np.full(8, math.pi / 2, np.float32)]))]).reshape(1, 48)

    T = 2048
    nb = half // T
    full = lambda shape: pl.BlockSpec(shape, lambda i: tuple(0 for _ in shape))

    def tc_call(g_half, ints_half):
        return pl.pallas_call(
            _tc_body,
            grid=(nb,),
            in_specs=[
                pl.BlockSpec((T, EMB), lambda i: (i, 0)),
                pl.BlockSpec((T, 8), lambda i: (i, 0)),
                full((EMB, EMB)),
                full((48, EMB)),
                full((1, 48)), full((1, 48)), full((1, 48)), full((1, 48)),
                full((3, EMB)),
                full((1, EMB)), full((1, EMB)), full((1, EMB)),
            ],
            out_specs=pl.BlockSpec((T, EMB), lambda i: (i, 0)),
            out_shape=jax.ShapeDtypeStruct((half, EMB), jnp.float32),
        )(g_half, ints_half, wc, wf, arow, brow, crow, drow,
          visit_table, b.reshape(1, EMB), gamma.reshape(1, EMB), beta.reshape(1, EMB))

    outs = [tc_call(gathered[h], ints[h * half:(h + 1) * half]) for h in range(nsplit)]
    return jnp.concatenate(outs, axis=0).reshape(B, L, EMB)


# speedup vs baseline: 1.7101x; 1.4795x over previous
import functools
import math

import jax
import jax.numpy as jnp
import numpy as np
from jax import lax
from jax.experimental import pallas as pl
from jax.experimental.pallas import tpu as pltpu

try:
    from jax.experimental.pallas import tpu_sc as plsc
except ImportError:
    plsc = None

EMB = 128
TEMB = 16
PE_MAX = 512
EPS = 1e-12


def _sc_gather(table, idx3d, n_tokens):
    info = plsc.get_sparse_core_info()
    nw = info.num_cores * info.num_subcores
    chunks_per_w = (n_tokens // 128) // nw
    per_w = chunks_per_w * 128
    mesh = plsc.VectorSubcoreMesh(core_axis_name="c", subcore_axis_name="s")

    nbuf = 5
    assert chunks_per_w % nbuf == 0, chunks_per_w

    @functools.partial(
        pl.kernel,
        mesh=mesh,
        out_type=jax.ShapeDtypeStruct((n_tokens, EMB), jnp.float32),
        scratch_types=[
            pltpu.VMEM((chunks_per_w, 128), jnp.int32),
            pltpu.VMEM((nbuf, 128, EMB), jnp.float32),
            pltpu.SemaphoreType.DMA((nbuf,)),
            pltpu.SemaphoreType.DMA((nbuf,)),
        ],
    )
    def gather_k(table_hbm, idx_hbm, out_hbm, idx_v, rows_v, sg, ss):
        wid = lax.axis_index("s") * info.num_cores + lax.axis_index("c")
        pltpu.sync_copy(idx_hbm.at[wid], idx_v)
        base = wid * per_w

        def g_start(j, b):
            pltpu.make_async_copy(table_hbm.at[idx_v.at[j]], rows_v.at[b],
                                  sg.at[b]).start()

        def g_wait(b):
            pltpu.make_async_copy(table_hbm.at[idx_v.at[0]], rows_v.at[b],
                                  sg.at[b]).wait()

        def s_start(j, b):
            pltpu.make_async_copy(rows_v.at[b],
                                  out_hbm.at[pl.ds(base + j * 128, 128)],
                                  ss.at[b]).start()

        def s_wait(b):
            pltpu.make_async_copy(rows_v.at[b], out_hbm.at[pl.ds(base, 128)],
                                  ss.at[b]).wait()

        for b in range(nbuf):
            g_start(b, b)

        def outer(t0, carry):
            for b in range(nbuf):
                j = t0 * nbuf + b
                g_wait(b)
                s_start(j, b)
                s_wait(b)
                nj = j + nbuf

                @pl.when(nj < chunks_per_w)
                def _():
                    g_start(nj, b)
            return carry

        lax.fori_loop(0, chunks_per_w // nbuf, outer, 0)

    return gather_k(table, idx3d)


def _fast_sin(x):
    inv_2pi = 0.15915494309189535
    c1 = 6.28125
    c2 = 0.0019353071795864769
    k = jnp.round(x * inv_2pi)
    r = (x - k * c1) - k * c2
    r2 = r * r
    p = 1.0 / 6227020800.0
    p = p * r2 - 1.0 / 39916800.0
    p = p * r2 + 1.0 / 362880.0
    p = p * r2 - 1.0 / 5040.0
    p = p * r2 + 1.0 / 120.0
    p = p * r2 - 1.0 / 6.0
    p = p * r2 + 1.0
    return r * p


def _tc_body(g_ref, ints_ref, wc_ref, wf_ref, a_ref, b48_ref, c_ref, d_ref,
             vt_ref, b_ref, gamma_ref, beta_ref, out_ref):
    ints = ints_ref[...]
    tsf = ints[0:1, :].astype(jnp.float32)
    agef = ints[1:2, :].astype(jnp.float32)
    normf = jnp.clip(ints[2:3, :] - ints[3:4, :], 0, PE_MAX - 1).astype(jnp.float32)
    arg = a_ref[...] * tsf + b48_ref[...] * agef + c_ref[...] * normf + d_ref[...]
    feat_t = _fast_sin(arg)
    vs = ints[4:5, :]
    row_ids = lax.broadcasted_iota(jnp.int32, (8, 1), 0)
    onehot_t = (vs == row_ids).astype(jnp.float32)
    acc = jnp.dot(g_ref[...], wc_ref[...], preferred_element_type=jnp.float32)
    acc = acc + jnp.dot(jnp.transpose(feat_t), wf_ref[...],
                        preferred_element_type=jnp.float32)
    acc = acc + b_ref[...]
    x = jnp.tanh(acc)
    x = x + jnp.dot(jnp.transpose(onehot_t), vt_ref[...],
                    preferred_element_type=jnp.float32)
    mu = jnp.mean(x, axis=-1, keepdims=True)
    var = jnp.mean(x * x, axis=-1, keepdims=True) - mu * mu
    out_ref[...] = ((x - mu) / jnp.sqrt(var + EPS)) * gamma_ref[...] + beta_ref[...]


def kernel(concept_ids, time_stamps, ages, visit_orders, visit_segments,
           concept_table, visit_table, w_time, phi_time, w_age, phi_age,
           pe, W, b, gamma, beta):
    B, L = concept_ids.shape
    BL = B * L
    nsplit = 1
    half = BL // nsplit

    idx4d = concept_ids.astype(jnp.int32).reshape(nsplit, 32, half // (32 * 128), 128)
    gathered = [_sc_gather(concept_table, idx4d[h], half) for h in range(nsplit)]

    i32 = jnp.int32
    ints_t = jnp.stack([
        time_stamps.astype(i32).reshape(BL),
        ages.astype(i32).reshape(BL),
        visit_orders.astype(i32).reshape(BL),
        jnp.broadcast_to(visit_orders[:, 0:1], (B, L)).astype(i32).reshape(BL),
        visit_segments.astype(i32).reshape(BL),
    ], axis=0)

    wc = W[:EMB]
    wf = jnp.concatenate([W[EMB:EMB + TEMB], W[EMB + TEMB:EMB + 2 * TEMB],
                          W[EMB + 2 * TEMB::2], W[EMB + 2 * TEMB + 1::2]], axis=0)
    div = np.exp(np.arange(0, TEMB, 2, dtype=np.float32)
                 * -(math.log(10000.0) / TEMB)).astype(np.float32)
    z8 = np.zeros(8, np.float32)
    z16 = np.zeros(16, np.float32)
    acol = jnp.concatenate([w_time[0], jnp.asarray(np.concatenate([z16, z8, z8]))]).reshape(48, 1)
    bcol = jnp.concatenate([jnp.asarray(z16), w_age[0], jnp.asarray(np.concatenate([z8, z8]))]).reshape(48, 1)
    ccol = jnp.asarray(np.concatenate([z16, z16, div, div])).reshape(48, 1)
    dcol = jnp.concatenate([phi_time[0], phi_age[0],
                            jnp.asarray(np.concatenate([z8, np.full(8, math.pi / 2, np.float32)]))]).reshape(48, 1)

    vt8 = jnp.concatenate([visit_table, jnp.zeros((5, EMB), jnp.float32)], axis=0)

    T = 1024
    nb = half // T
    full = lambda shape: pl.BlockSpec(shape, lambda i: tuple(0 for _ in shape))

    def tc_call(g_half, ints_half):
        return pl.pallas_call(
            _tc_body,
            grid=(nb,),
            in_specs=[
                pl.BlockSpec((T, EMB), lambda i: (i, 0)),
                pl.BlockSpec((5, T), lambda i: (0, i)),
                full((EMB, EMB)),
                full((48, EMB)),
                full((48, 1)), full((48, 1)), full((48, 1)), full((48, 1)),
                full((8, EMB)),
                full((1, EMB)), full((1, EMB)), full((1, EMB)),
            ],
            out_specs=pl.BlockSpec((T, EMB), lambda i: (i, 0)),
            out_shape=jax.ShapeDtypeStruct((half, EMB), jnp.float32),
        )(g_half, ints_half, wc, wf, acol, bcol, ccol, dcol,
          vt8, b.reshape(1, EMB), gamma.reshape(1, EMB), beta.reshape(1, EMB))

    outs = [tc_call(gathered[h], ints_t[:, h * half:(h + 1) * half]) for h in range(nsplit)]
    out = outs[0] if nsplit == 1 else jnp.concatenate(outs, axis=0)
    return out.reshape(B, L, EMB)


# speedup vs baseline: 1.7419x; 1.0186x over previous
import functools
import math

import jax
import jax.numpy as jnp
import numpy as np
from jax import lax
from jax.experimental import pallas as pl
from jax.experimental.pallas import tpu as pltpu

try:
    from jax.experimental.pallas import tpu_sc as plsc
except ImportError:
    plsc = None

EMB = 128
TEMB = 16
PE_MAX = 512
EPS = 1e-12


def _sc_gather(table, idx3d, n_tokens):
    dtype = table.dtype
    d = table.shape[1]
    info = plsc.get_sparse_core_info()
    nw = info.num_cores * info.num_subcores
    chunks_per_w = (n_tokens // 128) // nw
    per_w = chunks_per_w * 128
    mesh = plsc.VectorSubcoreMesh(core_axis_name="c", subcore_axis_name="s")

    nbuf = 5
    assert chunks_per_w % nbuf == 0, chunks_per_w

    @functools.partial(
        pl.kernel,
        mesh=mesh,
        out_type=jax.ShapeDtypeStruct((n_tokens, d), dtype),
        scratch_types=[
            pltpu.VMEM((chunks_per_w, 128), jnp.int32),
            pltpu.VMEM((nbuf, 128, d), dtype),
            pltpu.SemaphoreType.DMA((nbuf,)),
            pltpu.SemaphoreType.DMA((nbuf,)),
        ],
    )
    def gather_k(table_hbm, idx_hbm, out_hbm, idx_v, rows_v, sg, ss):
        wid = lax.axis_index("s") * info.num_cores + lax.axis_index("c")
        pltpu.sync_copy(idx_hbm.at[wid], idx_v)
        base = wid * per_w

        def g_start(j, b):
            pltpu.make_async_copy(table_hbm.at[idx_v.at[j]], rows_v.at[b],
                                  sg.at[b]).start()

        def g_wait(b):
            pltpu.make_async_copy(table_hbm.at[idx_v.at[0]], rows_v.at[b],
                                  sg.at[b]).wait()

        def s_start(j, b):
            pltpu.make_async_copy(rows_v.at[b],
                                  out_hbm.at[pl.ds(base + j * 128, 128)],
                                  ss.at[b]).start()

        def s_wait(b):
            pltpu.make_async_copy(rows_v.at[b], out_hbm.at[pl.ds(base, 128)],
                                  ss.at[b]).wait()

        for b in range(nbuf):
            g_start(b, b)

        def outer(t0, carry):
            for b in range(nbuf):
                j = t0 * nbuf + b
                g_wait(b)
                s_start(j, b)
                s_wait(b)
                nj = j + nbuf

                @pl.when(nj < chunks_per_w)
                def _():
                    g_start(nj, b)
            return carry

        lax.fori_loop(0, chunks_per_w // nbuf, outer, 0)

    return gather_k(table, idx3d)


def _fast_sin(x):
    inv_2pi = 0.15915494309189535
    c1 = 6.28125
    c2 = 0.0019353071795864769
    k = jnp.round(x * inv_2pi)
    r = (x - k * c1) - k * c2
    r2 = r * r
    p = 1.0 / 6227020800.0
    p = p * r2 - 1.0 / 39916800.0
    p = p * r2 + 1.0 / 362880.0
    p = p * r2 - 1.0 / 5040.0
    p = p * r2 + 1.0 / 120.0
    p = p * r2 - 1.0 / 6.0
    p = p * r2 + 1.0
    return r * p


def _tc_body(g_ref, ints_ref, wc_ref, wf_ref, a_ref, b48_ref, c_ref, d_ref,
             vt_ref, b_ref, gamma_ref, beta_ref, out_ref):
    ints = ints_ref[...]
    tsf = ints[0:1, :].astype(jnp.float32)
    agef = ints[1:2, :].astype(jnp.float32)
    normf = jnp.clip(ints[2:3, :] - ints[3:4, :], 0, PE_MAX - 1).astype(jnp.float32)
    arg = a_ref[...] * tsf + b48_ref[...] * agef + c_ref[...] * normf + d_ref[...]
    feat_t = _fast_sin(arg)
    vs = ints[4:5, :]
    row_ids = lax.broadcasted_iota(jnp.int32, (8, 1), 0)
    onehot_t = (vs == row_ids).astype(jnp.float32)
    acc = jnp.dot(g_ref[...], wc_ref[...], preferred_element_type=jnp.float32)
    acc = acc + jnp.dot(jnp.transpose(feat_t), wf_ref[...],
                        preferred_element_type=jnp.float32)
    acc = acc + b_ref[...]
    x = jnp.tanh(acc)
    x = x + jnp.dot(jnp.transpose(onehot_t), vt_ref[...],
                    preferred_element_type=jnp.float32)
    mu = jnp.mean(x, axis=-1, keepdims=True)
    var = jnp.mean(x * x, axis=-1, keepdims=True) - mu * mu
    scale = lax.rsqrt(var + EPS)
    out_ref[...] = ((x - mu) * scale) * gamma_ref[...] + beta_ref[...]


def kernel(concept_ids, time_stamps, ages, visit_orders, visit_segments,
           concept_table, visit_table, w_time, phi_time, w_age, phi_age,
           pe, W, b, gamma, beta):
    B, L = concept_ids.shape
    BL = B * L
    nsplit = 1
    half = BL // nsplit

    idx4d = concept_ids.astype(jnp.int32).reshape(nsplit, 32, half // (32 * 128), 128)
    gathered = [_sc_gather(concept_table, idx4d[h], half) for h in range(nsplit)]

    i32 = jnp.int32
    ints_t = jnp.stack([
        time_stamps.astype(i32).reshape(BL),
        ages.astype(i32).reshape(BL),
        visit_orders.astype(i32).reshape(BL),
        jnp.broadcast_to(visit_orders[:, 0:1], (B, L)).astype(i32).reshape(BL),
        visit_segments.astype(i32).reshape(BL),
    ], axis=0)

    wc = W[:EMB]
    wf = jnp.concatenate([W[EMB:EMB + TEMB], W[EMB + TEMB:EMB + 2 * TEMB],
                          W[EMB + 2 * TEMB::2], W[EMB + 2 * TEMB + 1::2]], axis=0)
    div = np.exp(np.arange(0, TEMB, 2, dtype=np.float32)
                 * -(math.log(10000.0) / TEMB)).astype(np.float32)
    z8 = np.zeros(8, np.float32)
    z16 = np.zeros(16, np.float32)
    acol = jnp.concatenate([w_time[0], jnp.asarray(np.concatenate([z16, z8, z8]))]).reshape(48, 1)
    bcol = jnp.concatenate([jnp.asarray(z16), w_age[0], jnp.asarray(np.concatenate([z8, z8]))]).reshape(48, 1)
    ccol = jnp.asarray(np.concatenate([z16, z16, div, div])).reshape(48, 1)
    dcol = jnp.concatenate([phi_time[0], phi_age[0],
                            jnp.asarray(np.concatenate([z8, np.full(8, math.pi / 2, np.float32)]))]).reshape(48, 1)

    vt8 = jnp.concatenate([visit_table, jnp.zeros((5, EMB), jnp.float32)], axis=0)

    T = 1024
    nb = half // T
    full = lambda shape: pl.BlockSpec(shape, lambda i: tuple(0 for _ in shape))

    def tc_call(g_half, ints_half):
        return pl.pallas_call(
            _tc_body,
            grid=(nb,),
            in_specs=[
                pl.BlockSpec((T, EMB), lambda i: (i, 0)),
                pl.BlockSpec((5, T), lambda i: (0, i)),
                full((EMB, EMB)),
                full((48, EMB)),
                full((48, 1)), full((48, 1)), full((48, 1)), full((48, 1)),
                full((8, EMB)),
                full((1, EMB)), full((1, EMB)), full((1, EMB)),
            ],
            out_specs=pl.BlockSpec((T, EMB), lambda i: (i, 0)),
            out_shape=jax.ShapeDtypeStruct((half, EMB), jnp.float32),
        )(g_half, ints_half, wc, wf, acol, bcol, ccol, dcol,
          vt8, b.reshape(1, EMB), gamma.reshape(1, EMB), beta.reshape(1, EMB))

    outs = [tc_call(gathered[h], ints_t[:, h * half:(h + 1) * half]) for h in range(nsplit)]
    out = outs[0] if nsplit == 1 else jnp.concatenate(outs, axis=0)
    return out.reshape(B, L, EMB)


# speedup vs baseline: 2.4342x; 1.3974x over previous
import functools
import math

import jax
import jax.numpy as jnp
import numpy as np
from jax import lax
from jax.experimental import pallas as pl
from jax.experimental.pallas import tpu as pltpu

try:
    from jax.experimental.pallas import tpu_sc as plsc
except ImportError:
    plsc = None

EMB = 128
TEMB = 16
PE_MAX = 512
EPS = 1e-12


def _sc_gather(table, idx3d, n_tokens):
    dtype = table.dtype
    d = table.shape[1]
    info = plsc.get_sparse_core_info()
    nw = info.num_cores * info.num_subcores
    chunks_per_w = (n_tokens // 128) // nw
    per_w = chunks_per_w * 128
    mesh = plsc.VectorSubcoreMesh(core_axis_name="c", subcore_axis_name="s")

    nbuf = 5
    assert chunks_per_w % nbuf == 0, chunks_per_w

    @functools.partial(
        pl.kernel,
        mesh=mesh,
        out_type=jax.ShapeDtypeStruct((n_tokens, d), dtype),
        scratch_types=[
            pltpu.VMEM((chunks_per_w, 128), jnp.int32),
            pltpu.VMEM((nbuf, 128, d), dtype),
            pltpu.SemaphoreType.DMA((nbuf,)),
            pltpu.SemaphoreType.DMA((nbuf,)),
        ],
    )
    def gather_k(table_hbm, idx_hbm, out_hbm, idx_v, rows_v, sg, ss):
        wid = lax.axis_index("s") * info.num_cores + lax.axis_index("c")
        pltpu.sync_copy(idx_hbm.at[wid], idx_v)
        base = wid * per_w

        def g_start(j, b):
            pltpu.make_async_copy(table_hbm.at[idx_v.at[j]], rows_v.at[b],
                                  sg.at[b]).start()

        def g_wait(b):
            pltpu.make_async_copy(table_hbm.at[idx_v.at[0]], rows_v.at[b],
                                  sg.at[b]).wait()

        def s_start(j, b):
            pltpu.make_async_copy(rows_v.at[b],
                                  out_hbm.at[pl.ds(base + j * 128, 128)],
                                  ss.at[b]).start()

        def s_wait(b):
            pltpu.make_async_copy(rows_v.at[b], out_hbm.at[pl.ds(base, 128)],
                                  ss.at[b]).wait()

        for b in range(nbuf):
            g_start(b, b)

        def outer(t0, carry):
            for b in range(nbuf):
                j = t0 * nbuf + b
                g_wait(b)
                s_start(j, b)
                s_wait(b)
                nj = j + nbuf

                @pl.when(nj < chunks_per_w)
                def _():
                    g_start(nj, b)
            return carry

        lax.fori_loop(0, chunks_per_w // nbuf, outer, 0)

    return gather_k(table, idx3d)


def _fast_sin(x):
    inv_2pi = 0.15915494309189535
    c1 = 6.28125
    c2 = 0.0019353071795864769
    k = jnp.round(x * inv_2pi)
    r = (x - k * c1) - k * c2
    r2 = r * r
    p = 1.0 / 6227020800.0
    p = p * r2 - 1.0 / 39916800.0
    p = p * r2 + 1.0 / 362880.0
    p = p * r2 - 1.0 / 5040.0
    p = p * r2 + 1.0 / 120.0
    p = p * r2 - 1.0 / 6.0
    p = p * r2 + 1.0
    return r * p


def _tc_body(g_ref, ints_ref, wc_ref, wf_ref, a_ref, b48_ref, c_ref, d_ref,
             vt_ref, b_ref, gamma_ref, beta_ref, out_ref):
    ints = ints_ref[...]
    tsf = ints[0:1, :].astype(jnp.float32)
    agef = ints[1:2, :].astype(jnp.float32)
    normf = jnp.clip(ints[2:3, :] - ints[3:4, :], 0, PE_MAX - 1).astype(jnp.float32)
    arg = a_ref[...] * tsf + b48_ref[...] * agef + c_ref[...] * normf + d_ref[...]
    feat_t = _fast_sin(arg)
    vs = ints[4:5, :]
    row_ids = lax.broadcasted_iota(jnp.int32, (8, 1), 0)
    onehot_t = (vs == row_ids).astype(jnp.float32)
    acc = jnp.dot(g_ref[...], wc_ref[...], preferred_element_type=jnp.float32)
    acc = acc + jnp.dot(jnp.transpose(feat_t), wf_ref[...],
                        preferred_element_type=jnp.float32)
    acc = acc + b_ref[...]
    x = jnp.tanh(acc)
    x = x + jnp.dot(jnp.transpose(onehot_t), vt_ref[...],
                    preferred_element_type=jnp.float32)
    mu = jnp.mean(x, axis=-1, keepdims=True)
    var = jnp.mean(x * x, axis=-1, keepdims=True) - mu * mu
    scale = lax.rsqrt(var + EPS)
    out_ref[...] = ((x - mu) * scale) * gamma_ref[...] + beta_ref[...]


def kernel(concept_ids, time_stamps, ages, visit_orders, visit_segments,
           concept_table, visit_table, w_time, phi_time, w_age, phi_age,
           pe, W, b, gamma, beta):
    B, L = concept_ids.shape
    BL = B * L
    nsplit = 1
    half = BL // nsplit

    idx4d = concept_ids.astype(jnp.int32).reshape(nsplit, 32, half // (32 * 128), 128)
    gathered = [_sc_gather(concept_table, idx4d[h], half) for h in range(nsplit)]

    i32 = jnp.int32
    ints_t = jnp.stack([
        time_stamps.astype(i32).reshape(BL),
        ages.astype(i32).reshape(BL),
        visit_orders.astype(i32).reshape(BL),
        jnp.broadcast_to(visit_orders[:, 0:1], (B, L)).astype(i32).reshape(BL),
        visit_segments.astype(i32).reshape(BL),
    ], axis=0)

    wc = W[:EMB]
    wf = jnp.concatenate([W[EMB:EMB + TEMB], W[EMB + TEMB:EMB + 2 * TEMB],
                          W[EMB + 2 * TEMB::2], W[EMB + 2 * TEMB + 1::2]], axis=0)
    div = np.exp(np.arange(0, TEMB, 2, dtype=np.float32)
                 * -(math.log(10000.0) / TEMB)).astype(np.float32)
    z8 = np.zeros(8, np.float32)
    z16 = np.zeros(16, np.float32)
    acol = jnp.concatenate([w_time[0], jnp.asarray(np.concatenate([z16, z8, z8]))]).reshape(48, 1)
    bcol = jnp.concatenate([jnp.asarray(z16), w_age[0], jnp.asarray(np.concatenate([z8, z8]))]).reshape(48, 1)
    ccol = jnp.asarray(np.concatenate([z16, z16, div, div])).reshape(48, 1)
    dcol = jnp.concatenate([phi_time[0], phi_age[0],
                            jnp.asarray(np.concatenate([z8, np.full(8, math.pi / 2, np.float32)]))]).reshape(48, 1)

    vt8 = jnp.concatenate([visit_table, jnp.zeros((5, EMB), jnp.float32)], axis=0)

    T = 4096
    nb = half // T
    full = lambda shape: pl.BlockSpec(shape, lambda i: tuple(0 for _ in shape))

    def tc_call(g_half, ints_half):
        return pl.pallas_call(
            _tc_body,
            grid=(nb,),
            in_specs=[
                pl.BlockSpec((T, EMB), lambda i: (i, 0)),
                pl.BlockSpec((5, T), lambda i: (0, i)),
                full((EMB, EMB)),
                full((48, EMB)),
                full((48, 1)), full((48, 1)), full((48, 1)), full((48, 1)),
                full((8, EMB)),
                full((1, EMB)), full((1, EMB)), full((1, EMB)),
            ],
            out_specs=pl.BlockSpec((T, EMB), lambda i: (i, 0)),
            out_shape=jax.ShapeDtypeStruct((half, EMB), jnp.float32),
        )(g_half, ints_half, wc, wf, acol, bcol, ccol, dcol,
          vt8, b.reshape(1, EMB), gamma.reshape(1, EMB), beta.reshape(1, EMB))

    outs = [tc_call(gathered[h], ints_t[:, h * half:(h + 1) * half]) for h in range(nsplit)]
    out = outs[0] if nsplit == 1 else jnp.concatenate(outs, axis=0)
    return out.reshape(B, L, EMB)


# speedup vs baseline: 2.4587x; 1.0101x over previous
import functools
import math

import jax
import jax.numpy as jnp
import numpy as np
from jax import lax
from jax.experimental import pallas as pl
from jax.experimental.pallas import tpu as pltpu

try:
    from jax.experimental.pallas import tpu_sc as plsc
except ImportError:
    plsc = None

EMB = 128
TEMB = 16
PE_MAX = 512
EPS = 1e-12


def _sc_gather(table, idx3d, n_tokens):
    dtype = table.dtype
    d = table.shape[1]
    info = plsc.get_sparse_core_info()
    nw = info.num_cores * info.num_subcores
    chunks_per_w = (n_tokens // 128) // nw
    per_w = chunks_per_w * 128
    mesh = plsc.VectorSubcoreMesh(core_axis_name="c", subcore_axis_name="s")

    nbuf = 5
    assert chunks_per_w % nbuf == 0, chunks_per_w

    @functools.partial(
        pl.kernel,
        mesh=mesh,
        out_type=jax.ShapeDtypeStruct((n_tokens, d), dtype),
        scratch_types=[
            pltpu.VMEM((chunks_per_w, 128), jnp.int32),
            pltpu.VMEM((nbuf, 128, d), dtype),
            pltpu.SemaphoreType.DMA((nbuf,)),
            pltpu.SemaphoreType.DMA((nbuf,)),
        ],
    )
    def gather_k(table_hbm, idx_hbm, out_hbm, idx_v, rows_v, sg, ss):
        wid = lax.axis_index("s") * info.num_cores + lax.axis_index("c")
        pltpu.sync_copy(idx_hbm.at[wid], idx_v)
        base = wid * per_w

        def g_start(j, b):
            pltpu.make_async_copy(table_hbm.at[idx_v.at[j]], rows_v.at[b],
                                  sg.at[b]).start()

        def g_wait(b):
            pltpu.make_async_copy(table_hbm.at[idx_v.at[0]], rows_v.at[b],
                                  sg.at[b]).wait()

        def s_start(j, b):
            pltpu.make_async_copy(rows_v.at[b],
                                  out_hbm.at[pl.ds(base + j * 128, 128)],
                                  ss.at[b]).start()

        def s_wait(b):
            pltpu.make_async_copy(rows_v.at[b], out_hbm.at[pl.ds(base, 128)],
                                  ss.at[b]).wait()

        for b in range(nbuf):
            g_start(b, b)

        def outer(t0, carry):
            for b in range(nbuf):
                j = t0 * nbuf + b
                g_wait(b)
                s_start(j, b)
                s_wait(b)
                nj = j + nbuf

                @pl.when(nj < chunks_per_w)
                def _():
                    g_start(nj, b)
            return carry

        lax.fori_loop(0, chunks_per_w // nbuf, outer, 0)

    return gather_k(table, idx3d)


def _fast_sin(x):
    inv_2pi = 0.15915494309189535
    c1 = 6.28125
    c2 = 0.0019353071795864769
    k = jnp.round(x * inv_2pi)
    r = (x - k * c1) - k * c2
    r2 = r * r
    p = 1.0 / 6227020800.0
    p = p * r2 - 1.0 / 39916800.0
    p = p * r2 + 1.0 / 362880.0
    p = p * r2 - 1.0 / 5040.0
    p = p * r2 + 1.0 / 120.0
    p = p * r2 - 1.0 / 6.0
    p = p * r2 + 1.0
    return r * p


def _tc_body(g_ref, ints_ref, wc_ref, wf_ref, a_ref, b48_ref, c_ref, d_ref,
             vt_ref, b_ref, gamma_ref, beta_ref, out_ref):
    ints = ints_ref[...]
    tsf = ints[0:1, :].astype(jnp.float32)
    agef = ints[1:2, :].astype(jnp.float32)
    normf = jnp.clip(ints[2:3, :] - ints[3:4, :], 0, PE_MAX - 1).astype(jnp.float32)
    arg = a_ref[...] * tsf + b48_ref[...] * agef + c_ref[...] * normf + d_ref[...]
    feat_t = _fast_sin(arg)
    vs = ints[4:5, :]
    row_ids = lax.broadcasted_iota(jnp.int32, (8, 1), 0)
    onehot_t = (vs == row_ids).astype(jnp.float32)
    acc = jnp.dot(g_ref[...], wc_ref[...], preferred_element_type=jnp.float32)
    acc = acc + jnp.dot(jnp.transpose(feat_t), wf_ref[...],
                        preferred_element_type=jnp.float32)
    acc = acc + b_ref[...]
    x = jnp.tanh(acc)
    x = x + jnp.dot(jnp.transpose(onehot_t), vt_ref[...],
                    preferred_element_type=jnp.float32)
    mu = jnp.mean(x, axis=-1, keepdims=True)
    var = jnp.mean(x * x, axis=-1, keepdims=True) - mu * mu
    scale = lax.rsqrt(var + EPS)
    out_ref[...] = ((x - mu) * scale) * gamma_ref[...] + beta_ref[...]


def kernel(concept_ids, time_stamps, ages, visit_orders, visit_segments,
           concept_table, visit_table, w_time, phi_time, w_age, phi_age,
           pe, W, b, gamma, beta):
    B, L = concept_ids.shape
    BL = B * L
    nsplit = 1
    half = BL // nsplit

    idx4d = concept_ids.astype(jnp.int32).reshape(nsplit, 32, half // (32 * 128), 128)
    gathered = [_sc_gather(concept_table, idx4d[h], half) for h in range(nsplit)]

    i32 = jnp.int32
    ints_t = jnp.stack([
        time_stamps.astype(i32).reshape(BL),
        ages.astype(i32).reshape(BL),
        visit_orders.astype(i32).reshape(BL),
        jnp.broadcast_to(visit_orders[:, 0:1], (B, L)).astype(i32).reshape(BL),
        visit_segments.astype(i32).reshape(BL),
    ], axis=0)

    wc = W[:EMB]
    wf = jnp.concatenate([W[EMB:EMB + TEMB], W[EMB + TEMB:EMB + 2 * TEMB],
                          W[EMB + 2 * TEMB::2], W[EMB + 2 * TEMB + 1::2]], axis=0)
    div = np.exp(np.arange(0, TEMB, 2, dtype=np.float32)
                 * -(math.log(10000.0) / TEMB)).astype(np.float32)
    z8 = np.zeros(8, np.float32)
    z16 = np.zeros(16, np.float32)
    acol = jnp.concatenate([w_time[0], jnp.asarray(np.concatenate([z16, z8, z8]))]).reshape(48, 1)
    bcol = jnp.concatenate([jnp.asarray(z16), w_age[0], jnp.asarray(np.concatenate([z8, z8]))]).reshape(48, 1)
    ccol = jnp.asarray(np.concatenate([z16, z16, div, div])).reshape(48, 1)
    dcol = jnp.concatenate([phi_time[0], phi_age[0],
                            jnp.asarray(np.concatenate([z8, np.full(8, math.pi / 2, np.float32)]))]).reshape(48, 1)

    vt8 = jnp.concatenate([visit_table, jnp.zeros((5, EMB), jnp.float32)], axis=0)

    T = 8192
    nb = half // T
    full = lambda shape: pl.BlockSpec(shape, lambda i: tuple(0 for _ in shape))

    def tc_call(g_half, ints_half):
        return pl.pallas_call(
            _tc_body,
            grid=(nb,),
            in_specs=[
                pl.BlockSpec((T, EMB), lambda i: (i, 0)),
                pl.BlockSpec((5, T), lambda i: (0, i)),
                full((EMB, EMB)),
                full((48, EMB)),
                full((48, 1)), full((48, 1)), full((48, 1)), full((48, 1)),
                full((8, EMB)),
                full((1, EMB)), full((1, EMB)), full((1, EMB)),
            ],
            out_specs=pl.BlockSpec((T, EMB), lambda i: (i, 0)),
            out_shape=jax.ShapeDtypeStruct((half, EMB), jnp.float32),
        )(g_half, ints_half, wc, wf, acol, bcol, ccol, dcol,
          vt8, b.reshape(1, EMB), gamma.reshape(1, EMB), beta.reshape(1, EMB))

    outs = [tc_call(gathered[h], ints_t[:, h * half:(h + 1) * half]) for h in range(nsplit)]
    out = outs[0] if nsplit == 1 else jnp.concatenate(outs, axis=0)
    return out.reshape(B, L, EMB)


# speedup vs baseline: 2.4599x; 1.0005x over previous
import functools
import math

import jax
import jax.numpy as jnp
import numpy as np
from jax import lax
from jax.experimental import pallas as pl
from jax.experimental.pallas import tpu as pltpu

try:
    from jax.experimental.pallas import tpu_sc as plsc
except ImportError:
    plsc = None

EMB = 128
TEMB = 16
PE_MAX = 512
EPS = 1e-12


def _sc_gather(table, idx3d, n_tokens):
    dtype = table.dtype
    d = table.shape[1]
    info = plsc.get_sparse_core_info()
    nw = info.num_cores * info.num_subcores
    rc = 64
    chunks_per_w = (n_tokens // rc) // nw
    per_w = chunks_per_w * rc
    mesh = plsc.VectorSubcoreMesh(core_axis_name="c", subcore_axis_name="s")

    nbuf = 10
    assert chunks_per_w % nbuf == 0, chunks_per_w

    @functools.partial(
        pl.kernel,
        mesh=mesh,
        out_type=jax.ShapeDtypeStruct((n_tokens, d), dtype),
        scratch_types=[
            pltpu.VMEM((chunks_per_w, rc), jnp.int32),
            pltpu.VMEM((nbuf, rc, d), dtype),
            pltpu.SemaphoreType.DMA((nbuf,)),
            pltpu.SemaphoreType.DMA((nbuf,)),
        ],
    )
    def gather_k(table_hbm, idx_hbm, out_hbm, idx_v, rows_v, sg, ss):
        wid = lax.axis_index("s") * info.num_cores + lax.axis_index("c")
        pltpu.sync_copy(idx_hbm.at[wid], idx_v)
        base = wid * per_w

        def g_start(j, b):
            pltpu.make_async_copy(table_hbm.at[idx_v.at[j]], rows_v.at[b],
                                  sg.at[b]).start()

        def g_wait(b):
            pltpu.make_async_copy(table_hbm.at[idx_v.at[0]], rows_v.at[b],
                                  sg.at[b]).wait()

        def s_start(j, b):
            pltpu.make_async_copy(rows_v.at[b],
                                  out_hbm.at[pl.ds(base + j * rc, rc)],
                                  ss.at[b]).start()

        def s_wait(b):
            pltpu.make_async_copy(rows_v.at[b], out_hbm.at[pl.ds(base, rc)],
                                  ss.at[b]).wait()

        for b in range(nbuf):
            g_start(b, b)

        def outer(t0, carry):
            for b in range(nbuf):
                j = t0 * nbuf + b
                g_wait(b)
                s_start(j, b)
                s_wait(b)
                nj = j + nbuf

                @pl.when(nj < chunks_per_w)
                def _():
                    g_start(nj, b)
            return carry

        lax.fori_loop(0, chunks_per_w // nbuf, outer, 0)

    return gather_k(table, idx3d)


def _fast_sin(x):
    inv_2pi = 0.15915494309189535
    c1 = 6.28125
    c2 = 0.0019353071795864769
    k = jnp.round(x * inv_2pi)
    r = (x - k * c1) - k * c2
    r2 = r * r
    p = 1.0 / 6227020800.0
    p = p * r2 - 1.0 / 39916800.0
    p = p * r2 + 1.0 / 362880.0
    p = p * r2 - 1.0 / 5040.0
    p = p * r2 + 1.0 / 120.0
    p = p * r2 - 1.0 / 6.0
    p = p * r2 + 1.0
    return r * p


def _tc_body(g_ref, ints_ref, wc_ref, wf_ref, a_ref, b48_ref, c_ref, d_ref,
             vt_ref, b_ref, gamma_ref, beta_ref, out_ref):
    ints = ints_ref[...]
    tsf = ints[0:1, :].astype(jnp.float32)
    agef = ints[1:2, :].astype(jnp.float32)
    normf = jnp.clip(ints[2:3, :] - ints[3:4, :], 0, PE_MAX - 1).astype(jnp.float32)
    arg = a_ref[...] * tsf + b48_ref[...] * agef + c_ref[...] * normf + d_ref[...]
    feat_t = _fast_sin(arg)
    vs = ints[4:5, :]
    row_ids = lax.broadcasted_iota(jnp.int32, (8, 1), 0)
    onehot_t = (vs == row_ids).astype(jnp.float32)
    acc = jnp.dot(g_ref[...], wc_ref[...], preferred_element_type=jnp.float32)
    acc = acc + jnp.dot(jnp.transpose(feat_t), wf_ref[...],
                        preferred_element_type=jnp.float32)
    acc = acc + b_ref[...]
    x = jnp.tanh(acc)
    x = x + jnp.dot(jnp.transpose(onehot_t), vt_ref[...],
                    preferred_element_type=jnp.float32)
    mu = jnp.mean(x, axis=-1, keepdims=True)
    var = jnp.mean(x * x, axis=-1, keepdims=True) - mu * mu
    scale = lax.rsqrt(var + EPS)
    out_ref[...] = ((x - mu) * scale) * gamma_ref[...] + beta_ref[...]


def kernel(concept_ids, time_stamps, ages, visit_orders, visit_segments,
           concept_table, visit_table, w_time, phi_time, w_age, phi_age,
           pe, W, b, gamma, beta):
    B, L = concept_ids.shape
    BL = B * L
    nsplit = 1
    half = BL // nsplit

    idx4d = concept_ids.astype(jnp.int32).reshape(nsplit, 32, half // (32 * 64), 64)
    gathered = [_sc_gather(concept_table, idx4d[h], half) for h in range(nsplit)]

    i32 = jnp.int32
    ints_t = jnp.stack([
        time_stamps.astype(i32).reshape(BL),
        ages.astype(i32).reshape(BL),
        visit_orders.astype(i32).reshape(BL),
        jnp.broadcast_to(visit_orders[:, 0:1], (B, L)).astype(i32).reshape(BL),
        visit_segments.astype(i32).reshape(BL),
    ], axis=0)

    wc = W[:EMB]
    wf = jnp.concatenate([W[EMB:EMB + TEMB], W[EMB + TEMB:EMB + 2 * TEMB],
                          W[EMB + 2 * TEMB::2], W[EMB + 2 * TEMB + 1::2]], axis=0)
    div = np.exp(np.arange(0, TEMB, 2, dtype=np.float32)
                 * -(math.log(10000.0) / TEMB)).astype(np.float32)
    z8 = np.zeros(8, np.float32)
    z16 = np.zeros(16, np.float32)
    acol = jnp.concatenate([w_time[0], jnp.asarray(np.concatenate([z16, z8, z8]))]).reshape(48, 1)
    bcol = jnp.concatenate([jnp.asarray(z16), w_age[0], jnp.asarray(np.concatenate([z8, z8]))]).reshape(48, 1)
    ccol = jnp.asarray(np.concatenate([z16, z16, div, div])).reshape(48, 1)
    dcol = jnp.concatenate([phi_time[0], phi_age[0],
                            jnp.asarray(np.concatenate([z8, np.full(8, math.pi / 2, np.float32)]))]).reshape(48, 1)

    vt8 = jnp.concatenate([visit_table, jnp.zeros((5, EMB), jnp.float32)], axis=0)

    T = 8192
    nb = half // T
    full = lambda shape: pl.BlockSpec(shape, lambda i: tuple(0 for _ in shape))

    def tc_call(g_half, ints_half):
        return pl.pallas_call(
            _tc_body,
            grid=(nb,),
            in_specs=[
                pl.BlockSpec((T, EMB), lambda i: (i, 0)),
                pl.BlockSpec((5, T), lambda i: (0, i)),
                full((EMB, EMB)),
                full((48, EMB)),
                full((48, 1)), full((48, 1)), full((48, 1)), full((48, 1)),
                full((8, EMB)),
                full((1, EMB)), full((1, EMB)), full((1, EMB)),
            ],
            out_specs=pl.BlockSpec((T, EMB), lambda i: (i, 0)),
            out_shape=jax.ShapeDtypeStruct((half, EMB), jnp.float32),
        )(g_half, ints_half, wc, wf, acol, bcol, ccol, dcol,
          vt8, b.reshape(1, EMB), gamma.reshape(1, EMB), beta.reshape(1, EMB))

    outs = [tc_call(gathered[h], ints_t[:, h * half:(h + 1) * half]) for h in range(nsplit)]
    out = outs[0] if nsplit == 1 else jnp.concatenate(outs, axis=0)
    return out.reshape(B, L, EMB)
